# Initial kernel scaffold; baseline (speedup 1.0000x reference)
import jax
import jax.numpy as jnp
from jax.experimental import pallas as pl


def kernel(x, coords, lens, Wq, Wk, Wv):
    raise NotImplementedError("write your pallas kernel here")


# speedup vs baseline: 4.2631x; 4.2631x over previous
import jax
import jax.numpy as jnp
from jax import lax
from jax.experimental import pallas as pl
from jax.experimental.pallas import tpu as pltpu

F = 768
H = 12
DH = 64
K = 16
S = 2048
B = 2

BM = 512
BQ = 256
BA = 512


def _proj_kernel(x_ref, wq_ref, wk_ref, wv_ref, q_ref, k_ref, v_ref, ks_ref):
    i = pl.program_id(0)
    x = x_ref[...]
    q_ref[...] = jnp.dot(x, wq_ref[...], preferred_element_type=jnp.float32)
    kk = jnp.dot(x, wk_ref[...], preferred_element_type=jnp.float32)
    k_ref[...] = kk
    v_ref[...] = jnp.dot(x, wv_ref[...], preferred_element_type=jnp.float32)

    @pl.when(i % (S // BM) == 0)
    def _():
        ks_ref[...] = jnp.zeros_like(ks_ref)

    ks_ref[...] += jnp.sum(kk, axis=0, keepdims=True) * (1.0 / S)


def _knn_kernel(lens_ref, cq_ref, ck_ref, out_ref):
    b = pl.program_id(0)
    i = pl.program_id(1)
    n = lens_ref[b]
    qidx = i * BQ + lax.broadcasted_iota(jnp.int32, (BQ, 1), 0)
    jidx = lax.broadcasted_iota(jnp.int32, (BQ, S), 1)

    d2 = jnp.zeros((BQ, S), jnp.float32)
    for c in range(3):
        diff = ck_ref[c:c + 1, :] - cq_ref[:, c:c + 1]
        d2 = d2 + diff * diff
    dist = jnp.sqrt(d2)

    vq = qidx < n
    vk = jidx < n
    both = vq & vk
    one = jnp.logical_xor(vq, vk)
    key = jnp.where(both, dist, jnp.where(one, jnp.float32(1e30),
                                          jnp.float32(2e30)))

    big_i = jnp.int32(2 ** 30)
    removed = jnp.float32(3e38)
    D = key
    for t in range(K + 1):
        m = jnp.min(D, axis=1, keepdims=True)
        sel = jnp.where(D == m, jidx, big_i)
        idx = jnp.min(sel, axis=1, keepdims=True)
        if t > 0:
            out_ref[:, t - 1:t] = idx
        D = jnp.where(jidx == idx, removed, D)


def _attn_kernel(lens_ref, q_ref, k_ref, v_ref, nb_ref, out_ref):
    b = pl.program_id(0)
    n = lens_ref[b]
    jidx = lax.broadcasted_iota(jnp.int32, (BA, S), 1)

    mask = jnp.zeros((BA, S), jnp.bool_)
    for t in range(K):
        mask = jnp.logical_or(mask, jidx == nb_ref[:, t:t + 1])
    mask = jnp.logical_and(mask, jidx < n)

    scale = DH ** -0.5
    neg_inf = jnp.float32(-jnp.inf)
    for h in range(H):
        qh = q_ref[:, h * DH:(h + 1) * DH]
        kh = k_ref[:, h * DH:(h + 1) * DH]
        vh = v_ref[:, h * DH:(h + 1) * DH]
        logits = lax.dot_general(qh, kh, (((1,), (1,)), ((), ())),
                                 preferred_element_type=jnp.float32) * scale
        logits = jnp.where(mask, logits, neg_inf)
        rmax = jnp.max(logits, axis=1, keepdims=True)
        rmax = jnp.where(rmax > neg_inf, rmax, 0.0)
        p = jnp.where(mask, jnp.exp(logits - rmax), 0.0)
        denom = jnp.sum(p, axis=1, keepdims=True)
        p = p / jnp.where(denom > 0, denom, 1.0)
        out_ref[:, h * DH:(h + 1) * DH] = jnp.dot(
            p, vh, preferred_element_type=jnp.float32)


def kernel(x, coords, lens, Wq, Wk, Wv):
    x2d = x.reshape(B * S, F)
    lens = lens.astype(jnp.int32)

    q2d, k2d, v2d, ksum = pl.pallas_call(
        _proj_kernel,
        grid=(B * S // BM,),
        in_specs=[
            pl.BlockSpec((BM, F), lambda i: (i, 0)),
            pl.BlockSpec((F, F), lambda i: (0, 0)),
            pl.BlockSpec((F, F), lambda i: (0, 0)),
            pl.BlockSpec((F, F), lambda i: (0, 0)),
        ],
        out_specs=[
            pl.BlockSpec((BM, F), lambda i: (i, 0)),
            pl.BlockSpec((BM, F), lambda i: (i, 0)),
            pl.BlockSpec((BM, F), lambda i: (i, 0)),
            pl.BlockSpec((None, 1, F), lambda i: (i // (S // BM), 0, 0)),
        ],
        out_shape=[
            jax.ShapeDtypeStruct((B * S, F), jnp.float32),
            jax.ShapeDtypeStruct((B * S, F), jnp.float32),
            jax.ShapeDtypeStruct((B * S, F), jnp.float32),
            jax.ShapeDtypeStruct((B, 1, F), jnp.float32),
        ],
    )(x2d, Wq.T, Wk.T, Wv.T)

    coords_t = jnp.swapaxes(coords, 1, 2)

    neigh = pl.pallas_call(
        _knn_kernel,
        grid=(B, S // BQ),
        in_specs=[
            pl.BlockSpec(memory_space=pltpu.SMEM),
            pl.BlockSpec((None, BQ, 3), lambda b, i: (b, i, 0)),
            pl.BlockSpec((None, 3, S), lambda b, i: (b, 0, 0)),
        ],
        out_specs=pl.BlockSpec((None, BQ, K), lambda b, i: (b, i, 0)),
        out_shape=jax.ShapeDtypeStruct((B, S, K), jnp.int32),
    )(lens, coords, coords_t)

    q3 = q2d.reshape(B, S, F)
    k3 = k2d.reshape(B, S, F)
    v3 = v2d.reshape(B, S, F)

    out = pl.pallas_call(
        _attn_kernel,
        grid=(B, S // BA),
        in_specs=[
            pl.BlockSpec(memory_space=pltpu.SMEM),
            pl.BlockSpec((None, BA, F), lambda b, i: (b, i, 0)),
            pl.BlockSpec((None, S, F), lambda b, i: (b, 0, 0)),
            pl.BlockSpec((None, S, F), lambda b, i: (b, 0, 0)),
            pl.BlockSpec((None, BA, K), lambda b, i: (b, i, 0)),
        ],
        out_specs=pl.BlockSpec((None, BA, F), lambda b, i: (b, i, 0)),
        out_shape=jax.ShapeDtypeStruct((B, S, F), jnp.float32),
    )(lens, q3, k3, v3, neigh)

    metric = ksum.reshape(B, H, DH)
    return (out, metric)
